# Initial kernel scaffold; baseline (speedup 1.0000x reference)
import functools

import jax
import jax.numpy as jnp
from jax import lax
from jax.experimental import pallas as pl
from jax.experimental.pallas import tpu as pltpu
from jax.experimental.pallas import tpu_sc as plsc

N = 10000
K = 128
P = 128
E = 320000

NC = 2
NS = 16
NW = NC * NS

N_PAD = 10240
ROWS_PER_TILE = N_PAD // NS
CHUNK = 128
EW = 10112
E_PAD = EW * NW
N_CHUNKS = EW // CHUNK

BR = 1280
GRID = N_PAD // BR


def _sc_body(src_hbm, dst_hbm, mu_hbm, agg_out, deg_out,
             didx_v, sidx_v, rows_v, ones_v, zrow_v, zdeg_v,
             agg_sh, deg_sh, sem):
    c = lax.axis_index("c")
    s = lax.axis_index("s")
    wid = s * NC + c

    def _fill_zrow(i, _):
        for j in range(P // 16):
            zrow_v[i, pl.ds(j * 16, 16)] = jnp.zeros((16,), jnp.float32)
        return 0
    lax.fori_loop(0, CHUNK, _fill_zrow, 0)
    def _fill_zdeg(i, _):
        zdeg_v[pl.ds(i * 16, 16)] = jnp.zeros((16,), jnp.float32)
        return 0
    lax.fori_loop(0, ROWS_PER_TILE // 16, _fill_zdeg, 0)
    for j in range(CHUNK // 16):
        ones_v[pl.ds(j * 16, 16)] = jnp.ones((16,), jnp.float32)

    row0 = s * ROWS_PER_TILE
    for r in range(ROWS_PER_TILE // CHUNK):
        pltpu.sync_copy(zrow_v, agg_sh.at[pl.ds(row0 + r * CHUNK, CHUNK)])
    pltpu.sync_copy(zdeg_v, deg_sh.at[pl.ds(row0, ROWS_PER_TILE)])
    plsc.subcore_barrier()

    ebase = wid * EW
    def _edge_chunk(k, _):
        b = ebase + k * CHUNK
        pltpu.sync_copy(dst_hbm.at[pl.ds(b, CHUNK)], didx_v)
        pltpu.sync_copy(src_hbm.at[pl.ds(b, CHUNK)], sidx_v)
        pltpu.async_copy(mu_hbm.at[didx_v], rows_v, sem).wait()
        pltpu.sync_copy(rows_v, agg_sh.at[sidx_v], add=True)
        pltpu.sync_copy(ones_v, deg_sh.at[sidx_v], add=True)
        return 0
    lax.fori_loop(0, N_CHUNKS, _edge_chunk, 0)
    plsc.subcore_barrier()

    for r in range(ROWS_PER_TILE // CHUNK):
        off = row0 + r * CHUNK
        pltpu.sync_copy(agg_sh.at[pl.ds(off, CHUNK)],
                        agg_out.at[c, pl.ds(off, CHUNK)])
    pltpu.sync_copy(deg_sh.at[pl.ds(row0, ROWS_PER_TILE)],
                    deg_out.at[c, pl.ds(row0, ROWS_PER_TILE)])


_sc_call = pl.kernel(
    _sc_body,
    out_type=[
        jax.ShapeDtypeStruct((NC, N_PAD, P), jnp.float32),
        jax.ShapeDtypeStruct((NC, N_PAD), jnp.float32),
    ],
    mesh=plsc.VectorSubcoreMesh(core_axis_name="c", subcore_axis_name="s"),
    scratch_types=[
        pltpu.VMEM((CHUNK,), jnp.int32),
        pltpu.VMEM((CHUNK,), jnp.int32),
        pltpu.VMEM((CHUNK, P), jnp.float32),
        pltpu.VMEM((CHUNK,), jnp.float32),
        pltpu.VMEM((CHUNK, P), jnp.float32),
        pltpu.VMEM((ROWS_PER_TILE,), jnp.float32),
        pltpu.VMEM_SHARED((N_PAD, P), jnp.float32),
        pltpu.VMEM_SHARED((N_PAD,), jnp.float32),
        pltpu.SemaphoreType.DMA,
    ],
)


def _mu_body(x_ref, sv_ref, tv_ref, t1s_ref, t1t_ref, t2T_ref, mu_ref):
    xb = x_ref[...]
    nrm = jnp.sqrt(jnp.sum(xb * xb, axis=1, keepdims=True))
    nrm = jnp.where(nrm == 0.0, 1.0, nrm)
    xn = xb / nrm
    lin = (sv_ref[...] * t1s_ref[...] + tv_ref[...] * t1t_ref[...]
           + jnp.dot(xn, t2T_ref[...], preferred_element_type=jnp.float32))
    mu_ref[...] = jnp.maximum(lin, 0.0)


_mu_call = pl.pallas_call(
    _mu_body,
    grid=(GRID,),
    in_specs=[
        pl.BlockSpec((BR, K), lambda i: (i, 0)),
        pl.BlockSpec((BR, 1), lambda i: (i, 0)),
        pl.BlockSpec((BR, 1), lambda i: (i, 0)),
        pl.BlockSpec((1, P), lambda i: (0, 0)),
        pl.BlockSpec((1, P), lambda i: (0, 0)),
        pl.BlockSpec((K, P), lambda i: (0, 0)),
    ],
    out_specs=pl.BlockSpec((BR, P), lambda i: (i, 0)),
    out_shape=jax.ShapeDtypeStruct((N_PAD, P), jnp.float32),
)


def _b1_body(mu_ref, a0_ref, a1_ref, degt_ref, hA_ref, hB_ref,
             mp_ref, acc_ref):
    mu_b = mu_ref[...]
    deg = degt_ref[:, 0:1] + degt_ref[:, 1:2]
    sres = deg * mu_b - a0_ref[0] - a1_ref[0]
    mp = (jnp.dot(jnp.maximum(mu_b, 0.0), hA_ref[...],
                  preferred_element_type=jnp.float32)
          + jnp.dot(jnp.maximum(sres, 0.0), hB_ref[...],
                    preferred_element_type=jnp.float32))
    mp_ref[...] = mp
    part = jnp.sum(mp, axis=0, keepdims=True)

    @pl.when(pl.program_id(0) == 0)
    def _():
        acc_ref[...] = part

    @pl.when(pl.program_id(0) > 0)
    def _():
        acc_ref[...] += part


_b1_call = pl.pallas_call(
    _b1_body,
    grid=(GRID,),
    in_specs=[
        pl.BlockSpec((BR, P), lambda i: (i, 0)),
        pl.BlockSpec((1, BR, P), lambda i: (0, i, 0)),
        pl.BlockSpec((1, BR, P), lambda i: (1, i, 0)),
        pl.BlockSpec((BR, NC), lambda i: (i, 0)),
        pl.BlockSpec((P, P), lambda i: (0, 0)),
        pl.BlockSpec((P, P), lambda i: (0, 0)),
    ],
    out_specs=[
        pl.BlockSpec((BR, P), lambda i: (i, 0)),
        pl.BlockSpec((1, P), lambda i: (0, 0)),
    ],
    out_shape=[
        jax.ShapeDtypeStruct((N_PAD, P), jnp.float32),
        jax.ShapeDtypeStruct((1, P), jnp.float32),
    ],
)


def _b2_body(mp_ref, smu_ref, t4T_ref, t5T_ref, t3u_ref, t3d_ref, q_ref):
    up = jnp.dot(smu_ref[...], t4T_ref[...],
                 preferred_element_type=jnp.float32)
    cval = jnp.sum(jnp.maximum(up, 0.0) * t3u_ref[...])
    down = jnp.dot(mp_ref[...], t5T_ref[...],
                   preferred_element_type=jnp.float32)
    q_ref[...] = jnp.dot(jnp.maximum(down, 0.0), t3d_ref[...],
                         preferred_element_type=jnp.float32) + cval


_b2_call = pl.pallas_call(
    _b2_body,
    grid=(GRID,),
    in_specs=[
        pl.BlockSpec((BR, P), lambda i: (i, 0)),
        pl.BlockSpec((1, P), lambda i: (0, 0)),
        pl.BlockSpec((P, P), lambda i: (0, 0)),
        pl.BlockSpec((P, P), lambda i: (0, 0)),
        pl.BlockSpec((1, P), lambda i: (0, 0)),
        pl.BlockSpec((P, 1), lambda i: (0, 0)),
    ],
    out_specs=pl.BlockSpec((BR, 1), lambda i: (i, 0)),
    out_shape=jax.ShapeDtypeStruct((N_PAD, 1), jnp.float32),
)


def kernel(x, s_v, t_v, edge_index, theta1, theta2, theta3, theta4, theta5,
           h_theta):
    f32 = jnp.float32
    pad_n = N_PAD - N
    x_p = jnp.concatenate([x.astype(f32), jnp.zeros((pad_n, K), f32)], axis=0)
    sv_p = jnp.concatenate([s_v.astype(f32), jnp.zeros((pad_n,), f32)])[:, None]
    tv_p = jnp.concatenate([t_v.astype(f32), jnp.zeros((pad_n,), f32)])[:, None]

    src = edge_index[0].astype(jnp.int32)
    dst = edge_index[1].astype(jnp.int32)
    pad_e = E_PAD - E
    src_p = jnp.concatenate([src, jnp.full((pad_e,), N, jnp.int32)])
    dst_p = jnp.concatenate([dst, jnp.full((pad_e,), N, jnp.int32)])

    t1s = theta1[:, 0][None, :]
    t1t = theta1[:, 1][None, :]
    t2T = theta2.T
    hA = h_theta[:, :P].T
    hB = h_theta[:, P:].T
    t4T = theta4.T
    t5T = theta5.T
    t3u = theta3[:P, 0][None, :]
    t3d = theta3[P:, :]

    mu = _mu_call(x_p, sv_p, tv_p, t1s, t1t, t2T)
    agg2, deg2 = _sc_call(src_p, dst_p, mu)
    degt = deg2.T
    mp, smu = _b1_call(mu, agg2, agg2, degt, hA, hB)
    q = _b2_call(mp, smu, t4T, t5T, t3u, t3d)
    return q[:N]


# speedup vs baseline: 4.3672x; 4.3672x over previous
import functools

import jax
import jax.numpy as jnp
from jax import lax
from jax.experimental import pallas as pl
from jax.experimental.pallas import tpu as pltpu
from jax.experimental.pallas import tpu_sc as plsc

N = 10000
K = 128
P = 128
E = 320000

NC = 2
NS = 16
NW = NC * NS

N_PAD = 10240
ROWS_PER_TILE = N_PAD // NS
CHUNK = 128
EW = 10112
E_PAD = EW * NW
N_CHUNKS = EW // CHUNK

BR = 1280
GRID = N_PAD // BR


def _sc_body(src_hbm, dst_hbm, mu_hbm, agg_out, deg_out,
             didx_v, sidx_v, rows_v, ones_v, zrow_v, zdeg_v,
             agg_sh, deg_sh, sem):
    c = lax.axis_index("c")
    s = lax.axis_index("s")
    wid = s * NC + c

    def _fill_zrow(i, _):
        for j in range(P // 16):
            zrow_v[i, pl.ds(j * 16, 16)] = jnp.zeros((16,), jnp.float32)
        return 0
    lax.fori_loop(0, CHUNK, _fill_zrow, 0)
    def _fill_zdeg(i, _):
        zdeg_v[pl.ds(i * 16, 16)] = jnp.zeros((16,), jnp.float32)
        return 0
    lax.fori_loop(0, ROWS_PER_TILE // 16, _fill_zdeg, 0)
    for j in range(CHUNK // 16):
        ones_v[pl.ds(j * 16, 16)] = jnp.ones((16,), jnp.float32)

    row0 = s * ROWS_PER_TILE
    for r in range(ROWS_PER_TILE // CHUNK):
        pltpu.sync_copy(zrow_v, agg_sh.at[pl.ds(row0 + r * CHUNK, CHUNK)])
    pltpu.sync_copy(zdeg_v, deg_sh.at[pl.ds(row0, ROWS_PER_TILE)])
    plsc.subcore_barrier()

    ebase = wid * EW
    def _edge_chunk(k, _):
        b = ebase + k * CHUNK
        pltpu.sync_copy(dst_hbm.at[pl.ds(b, CHUNK)], didx_v)
        pltpu.sync_copy(src_hbm.at[pl.ds(b, CHUNK)], sidx_v)
        pltpu.async_copy(mu_hbm.at[didx_v], rows_v, sem).wait()
        pltpu.sync_copy(rows_v, agg_sh.at[sidx_v], add=True)
        pltpu.sync_copy(ones_v, deg_sh.at[sidx_v], add=True)
        return 0
    lax.fori_loop(0, N_CHUNKS, _edge_chunk, 0)
    plsc.subcore_barrier()

    for r in range(ROWS_PER_TILE // CHUNK):
        off = row0 + r * CHUNK
        pltpu.sync_copy(agg_sh.at[pl.ds(off, CHUNK)],
                        agg_out.at[c, pl.ds(off, CHUNK)])
    pltpu.sync_copy(deg_sh.at[pl.ds(row0, ROWS_PER_TILE)],
                    deg_out.at[c, pl.ds(row0, ROWS_PER_TILE)])


@functools.cache
def _get_sc_call():
  return pl.kernel(
    _sc_body,
    out_type=[
        jax.ShapeDtypeStruct((NC, N_PAD, P), jnp.float32),
        jax.ShapeDtypeStruct((NC, N_PAD), jnp.float32),
    ],
    mesh=plsc.VectorSubcoreMesh(core_axis_name="c", subcore_axis_name="s",
                                num_cores=NC, num_subcores=NS),
    scratch_types=[
        pltpu.VMEM((CHUNK,), jnp.int32),
        pltpu.VMEM((CHUNK,), jnp.int32),
        pltpu.VMEM((CHUNK, P), jnp.float32),
        pltpu.VMEM((CHUNK,), jnp.float32),
        pltpu.VMEM((CHUNK, P), jnp.float32),
        pltpu.VMEM((ROWS_PER_TILE,), jnp.float32),
        pltpu.VMEM_SHARED((N_PAD, P), jnp.float32),
        pltpu.VMEM_SHARED((N_PAD,), jnp.float32),
        pltpu.SemaphoreType.DMA,
    ],
  )


def _mu_body(x_ref, sv_ref, tv_ref, t1s_ref, t1t_ref, t2T_ref, mu_ref):
    xb = x_ref[...]
    nrm = jnp.sqrt(jnp.sum(xb * xb, axis=1, keepdims=True))
    nrm = jnp.where(nrm == 0.0, 1.0, nrm)
    xn = xb / nrm
    lin = (sv_ref[...] * t1s_ref[...] + tv_ref[...] * t1t_ref[...]
           + jnp.dot(xn, t2T_ref[...], preferred_element_type=jnp.float32))
    mu_ref[...] = jnp.maximum(lin, 0.0)


def _make_mu_call(interpret=False):
  return pl.pallas_call(
    _mu_body,
    grid=(GRID,),
    in_specs=[
        pl.BlockSpec((BR, K), lambda i: (i, 0)),
        pl.BlockSpec((BR, 1), lambda i: (i, 0)),
        pl.BlockSpec((BR, 1), lambda i: (i, 0)),
        pl.BlockSpec((1, P), lambda i: (0, 0)),
        pl.BlockSpec((1, P), lambda i: (0, 0)),
        pl.BlockSpec((K, P), lambda i: (0, 0)),
    ],
    out_specs=pl.BlockSpec((BR, P), lambda i: (i, 0)),
    out_shape=jax.ShapeDtypeStruct((N_PAD, P), jnp.float32),
    interpret=interpret,
  )


_mu_call = _make_mu_call()


def _b1_body(mu_ref, a0_ref, a1_ref, degt_ref, hA_ref, hB_ref,
             mp_ref, acc_ref):
    mu_b = mu_ref[...]
    deg = degt_ref[:, 0:1] + degt_ref[:, 1:2]
    sres = deg * mu_b - a0_ref[0] - a1_ref[0]
    mp = (jnp.dot(jnp.maximum(mu_b, 0.0), hA_ref[...],
                  preferred_element_type=jnp.float32)
          + jnp.dot(jnp.maximum(sres, 0.0), hB_ref[...],
                    preferred_element_type=jnp.float32))
    mp_ref[...] = mp
    part = jnp.sum(mp, axis=0, keepdims=True)

    @pl.when(pl.program_id(0) == 0)
    def _():
        acc_ref[...] = part

    @pl.when(pl.program_id(0) > 0)
    def _():
        acc_ref[...] += part


def _make_b1_call(interpret=False):
  return pl.pallas_call(
    _b1_body,
    grid=(GRID,),
    in_specs=[
        pl.BlockSpec((BR, P), lambda i: (i, 0)),
        pl.BlockSpec((1, BR, P), lambda i: (0, i, 0)),
        pl.BlockSpec((1, BR, P), lambda i: (1, i, 0)),
        pl.BlockSpec((BR, NC), lambda i: (i, 0)),
        pl.BlockSpec((P, P), lambda i: (0, 0)),
        pl.BlockSpec((P, P), lambda i: (0, 0)),
    ],
    out_specs=[
        pl.BlockSpec((BR, P), lambda i: (i, 0)),
        pl.BlockSpec((1, P), lambda i: (0, 0)),
    ],
    out_shape=[
        jax.ShapeDtypeStruct((N_PAD, P), jnp.float32),
        jax.ShapeDtypeStruct((1, P), jnp.float32),
    ],
    interpret=interpret,
  )


_b1_call = _make_b1_call()


def _b2_body(mp_ref, smu_ref, t4T_ref, t5T_ref, t3u_ref, t3d_ref, q_ref):
    up = jnp.dot(smu_ref[...], t4T_ref[...],
                 preferred_element_type=jnp.float32)
    cval = jnp.sum(jnp.maximum(up, 0.0) * t3u_ref[...])
    down = jnp.dot(mp_ref[...], t5T_ref[...],
                   preferred_element_type=jnp.float32)
    q_ref[...] = jnp.dot(jnp.maximum(down, 0.0), t3d_ref[...],
                         preferred_element_type=jnp.float32) + cval


def _make_b2_call(interpret=False):
  return pl.pallas_call(
    _b2_body,
    grid=(GRID,),
    in_specs=[
        pl.BlockSpec((BR, P), lambda i: (i, 0)),
        pl.BlockSpec((1, P), lambda i: (0, 0)),
        pl.BlockSpec((P, P), lambda i: (0, 0)),
        pl.BlockSpec((P, P), lambda i: (0, 0)),
        pl.BlockSpec((1, P), lambda i: (0, 0)),
        pl.BlockSpec((P, 1), lambda i: (0, 0)),
    ],
    out_specs=pl.BlockSpec((BR, 1), lambda i: (i, 0)),
    out_shape=jax.ShapeDtypeStruct((N_PAD, 1), jnp.float32),
    interpret=interpret,
  )


_b2_call = _make_b2_call()


def kernel(x, s_v, t_v, edge_index, theta1, theta2, theta3, theta4, theta5,
           h_theta):
    f32 = jnp.float32
    pad_n = N_PAD - N
    x_p = jnp.concatenate([x.astype(f32), jnp.zeros((pad_n, K), f32)], axis=0)
    sv_p = jnp.concatenate([s_v.astype(f32), jnp.zeros((pad_n,), f32)])[:, None]
    tv_p = jnp.concatenate([t_v.astype(f32), jnp.zeros((pad_n,), f32)])[:, None]

    src = edge_index[0].astype(jnp.int32)
    dst = edge_index[1].astype(jnp.int32)
    pad_e = E_PAD - E
    src_p = jnp.concatenate([src, jnp.full((pad_e,), N, jnp.int32)])
    dst_p = jnp.concatenate([dst, jnp.full((pad_e,), N, jnp.int32)])

    t1s = theta1[:, 0][None, :]
    t1t = theta1[:, 1][None, :]
    t2T = theta2.T
    hA = h_theta[:, :P].T
    hB = h_theta[:, P:].T
    t4T = theta4.T
    t5T = theta5.T
    t3u = theta3[:P, 0][None, :]
    t3d = theta3[P:, :]

    mu = _mu_call(x_p, sv_p, tv_p, t1s, t1t, t2T)
    agg2, deg2 = _get_sc_call()(src_p, dst_p, mu)
    degt = deg2.T
    mp, smu = _b1_call(mu, agg2, agg2, degt, hA, hB)
    q = _b2_call(mp, smu, t4T, t5T, t3u, t3d)
    return q[:N]


# speedup vs baseline: 10.6574x; 2.4403x over previous
import functools

import jax
import jax.numpy as jnp
from jax import lax
from jax.experimental import pallas as pl
from jax.experimental.pallas import tpu as pltpu
from jax.experimental.pallas import tpu_sc as plsc

N = 10000
K = 128
P = 128
E = 320000

NC = 2
NS = 16
NW = NC * NS

CHUNK = 128
N_CHUNKS = 79
EW = N_CHUNKS * CHUNK
E_PAD = EW * NW
L = 16

N_SP = 10112
ROWS_PER_TILE = N_SP // NS
DEG_SP = 10240
DEG_SLICE = DEG_SP // NS

BR_A = 2000
GRID = N // BR_A


def _sc_body(comb_hbm, mu_hbm, agg_out, deg_out, *refs):
    (c0, c1, d0, d1, x0, x1, r0, r1,
     ones_v, zrow_v, zdeg_v, agg_sh, deg_sh, g0, g1) = refs
    comb = (c0, c1)
    didx = (d0, d1)
    sidx = (x0, x1)
    rows = (r0, r1)
    gsem = (g0, g1)

    c = lax.axis_index("c")
    s = lax.axis_index("s")
    wid = s * NC + c
    ebase = wid * EW

    def _load_idx(k, b):
        pltpu.sync_copy(comb_hbm.at[pl.ds(ebase + k * CHUNK, CHUNK)],
                        comb[b])
        for j in range(CHUNK // L):
            w = comb[b][pl.ds(j * L, L)]
            didx[b][pl.ds(j * L, L)] = w >> 16
            sidx[b][pl.ds(j * L, L)] = w & 0xFFFF

    def _issue_gather(b):
        pltpu.async_copy(mu_hbm.at[didx[b]], rows[b], gsem[b])

    def _wait_gather(b):
        pltpu.make_async_copy(mu_hbm.at[didx[b]], rows[b], gsem[b]).wait()

    def _sync_scatter(b):
        pltpu.sync_copy(rows[b], agg_sh.at[sidx[b]], add=True)
        pltpu.sync_copy(ones_v, deg_sh.at[sidx[b]], add=True)

    def _fill_zrow(i, _):
        for j in range(P // L):
            zrow_v[i, pl.ds(j * L, L)] = jnp.zeros((L,), jnp.float32)
        return 0
    lax.fori_loop(0, 8, _fill_zrow, 0)
    for j in range(CHUNK // L):
        ones_v[pl.ds(j * L, L)] = jnp.ones((L,), jnp.float32)

    row0 = pl.multiple_of(s * ROWS_PER_TILE, 8)
    def _zero(r, _):
        off = pl.multiple_of(row0 + r * 8, 8)
        pltpu.sync_copy(zrow_v, agg_sh.at[pl.ds(off, 8)])
        return 0
    lax.fori_loop(0, ROWS_PER_TILE // 8, _zero, 0)
    def _fill_zdeg(i, _):
        o = pl.multiple_of(i * L, L)
        zdeg_v[pl.ds(o, L)] = jnp.zeros((L,), jnp.float32)
        return 0
    lax.fori_loop(0, DEG_SLICE // L, _fill_zdeg, 0)
    dgrow0 = pl.multiple_of(s * DEG_SLICE, 128)
    pltpu.sync_copy(zdeg_v, deg_sh.at[pl.ds(dgrow0, DEG_SLICE)])
    plsc.subcore_barrier()

    _load_idx(0, 0)
    _issue_gather(0)

    def _pair(i, _):
        for b in (0, 1):
            k = i * 2 + b
            bn = 1 - b
            _load_idx(k + 1, bn)
            _issue_gather(bn)
            _wait_gather(b)
            _sync_scatter(b)
        return 0
    lax.fori_loop(0, (N_CHUNKS - 1) // 2, _pair, 0)

    _wait_gather(0)
    _sync_scatter(0)
    plsc.subcore_barrier()

    def _out(r, _):
        off = pl.multiple_of(row0 + r * CHUNK, 8)
        pltpu.sync_copy(agg_sh.at[pl.ds(off, CHUNK)],
                        agg_out.at[c, pl.ds(off, CHUNK)])
        return 0
    lax.fori_loop(0, ROWS_PER_TILE // CHUNK, _out, 0)
    if ROWS_PER_TILE % CHUNK:
        offt = pl.multiple_of(row0 + (ROWS_PER_TILE // CHUNK) * CHUNK, 8)
        pltpu.sync_copy(agg_sh.at[pl.ds(offt, ROWS_PER_TILE % CHUNK)],
                        agg_out.at[c, pl.ds(offt, ROWS_PER_TILE % CHUNK)])
    dflat0 = pl.multiple_of(c * DEG_SP + dgrow0, 128)
    pltpu.sync_copy(deg_sh.at[pl.ds(dgrow0, DEG_SLICE)],
                    deg_out.at[pl.ds(dflat0, DEG_SLICE)])


@functools.cache
def _get_sc_call():
  return pl.kernel(
    _sc_body,
    out_type=[
        jax.ShapeDtypeStruct((NC, N_SP, P), jnp.float32),
        jax.ShapeDtypeStruct((NC * DEG_SP,), jnp.float32),
    ],
    mesh=plsc.VectorSubcoreMesh(core_axis_name="c", subcore_axis_name="s",
                                num_cores=NC, num_subcores=NS),
    scratch_types=(
        [pltpu.VMEM((CHUNK,), jnp.int32) for _ in range(6)]
        + [pltpu.VMEM((CHUNK, P), jnp.float32) for _ in range(2)]
        + [
            pltpu.VMEM((CHUNK,), jnp.float32),
            pltpu.VMEM((8, P), jnp.float32),
            pltpu.VMEM((DEG_SLICE,), jnp.float32),
            pltpu.VMEM_SHARED((N_SP, P), jnp.float32),
            pltpu.VMEM_SHARED((DEG_SP,), jnp.float32),
        ]
        + [pltpu.SemaphoreType.DMA for _ in range(2)]
    ),
  )


def _mu_body(x_ref, sv_ref, tv_ref, t1s_ref, t1t_ref, t2T_ref, mu_ref):
    xb = x_ref[...]
    nrm = jnp.sqrt(jnp.sum(xb * xb, axis=1, keepdims=True))
    nrm = jnp.where(nrm == 0.0, 1.0, nrm)
    xn = xb / nrm
    lin = (sv_ref[...] * t1s_ref[...] + tv_ref[...] * t1t_ref[...]
           + jnp.dot(xn, t2T_ref[...], preferred_element_type=jnp.float32))
    mu_ref[...] = jnp.maximum(lin, 0.0)


def _make_mu_call(interpret=False):
  return pl.pallas_call(
    _mu_body,
    grid=(GRID,),
    in_specs=[
        pl.BlockSpec((BR_A, K), lambda i: (i, 0)),
        pl.BlockSpec((BR_A, 1), lambda i: (i, 0)),
        pl.BlockSpec((BR_A, 1), lambda i: (i, 0)),
        pl.BlockSpec((1, P), lambda i: (0, 0)),
        pl.BlockSpec((1, P), lambda i: (0, 0)),
        pl.BlockSpec((K, P), lambda i: (0, 0)),
    ],
    out_specs=pl.BlockSpec((BR_A, P), lambda i: (i, 0)),
    out_shape=jax.ShapeDtypeStruct((N, P), jnp.float32),
    interpret=interpret,
  )


_mu_call = _make_mu_call()


def _b1_body(mu_ref, a0_ref, a1_ref, d0_ref, d1_ref, hA_ref, hB_ref,
             mp_ref, acc_ref):
    mu_b = mu_ref[...]
    deg = d0_ref[...] + d1_ref[...]
    sres = deg * mu_b - a0_ref[0] - a1_ref[0]
    mp = (jnp.dot(jnp.maximum(mu_b, 0.0), hA_ref[...],
                  preferred_element_type=jnp.float32)
          + jnp.dot(jnp.maximum(sres, 0.0), hB_ref[...],
                    preferred_element_type=jnp.float32))
    mp_ref[...] = mp
    part = jnp.sum(mp, axis=0, keepdims=True)

    @pl.when(pl.program_id(0) == 0)
    def _():
        acc_ref[...] = part

    @pl.when(pl.program_id(0) > 0)
    def _():
        acc_ref[...] += part


def _make_b1_call(interpret=False):
  return pl.pallas_call(
    _b1_body,
    grid=(GRID,),
    in_specs=[
        pl.BlockSpec((BR_A, P), lambda i: (i, 0)),
        pl.BlockSpec((1, BR_A, P), lambda i: (0, i, 0)),
        pl.BlockSpec((1, BR_A, P), lambda i: (1, i, 0)),
        pl.BlockSpec((BR_A, 1), lambda i: (i, 0)),
        pl.BlockSpec((BR_A, 1), lambda i: (i, 0)),
        pl.BlockSpec((P, P), lambda i: (0, 0)),
        pl.BlockSpec((P, P), lambda i: (0, 0)),
    ],
    out_specs=[
        pl.BlockSpec((BR_A, P), lambda i: (i, 0)),
        pl.BlockSpec((1, P), lambda i: (0, 0)),
    ],
    out_shape=[
        jax.ShapeDtypeStruct((N, P), jnp.float32),
        jax.ShapeDtypeStruct((1, P), jnp.float32),
    ],
    interpret=interpret,
  )


_b1_call = _make_b1_call()


def _b2_body(mp_ref, smu_ref, t4T_ref, t5T_ref, t3u_ref, t3d_ref, q_ref):
    up = jnp.dot(smu_ref[...], t4T_ref[...],
                 preferred_element_type=jnp.float32)
    cval = jnp.sum(jnp.maximum(up, 0.0) * t3u_ref[...])
    down = jnp.dot(mp_ref[...], t5T_ref[...],
                   preferred_element_type=jnp.float32)
    q_ref[...] = jnp.dot(jnp.maximum(down, 0.0), t3d_ref[...],
                         preferred_element_type=jnp.float32) + cval


def _make_b2_call(interpret=False):
  return pl.pallas_call(
    _b2_body,
    grid=(GRID,),
    in_specs=[
        pl.BlockSpec((BR_A, P), lambda i: (i, 0)),
        pl.BlockSpec((1, P), lambda i: (0, 0)),
        pl.BlockSpec((P, P), lambda i: (0, 0)),
        pl.BlockSpec((P, P), lambda i: (0, 0)),
        pl.BlockSpec((1, P), lambda i: (0, 0)),
        pl.BlockSpec((P, 1), lambda i: (0, 0)),
    ],
    out_specs=pl.BlockSpec((BR_A, 1), lambda i: (i, 0)),
    out_shape=jax.ShapeDtypeStruct((N, 1), jnp.float32),
    interpret=interpret,
  )


_b2_call = _make_b2_call()


def kernel(x, s_v, t_v, edge_index, theta1, theta2, theta3, theta4, theta5,
           h_theta):
    src = edge_index[0].astype(jnp.int32)
    dst = edge_index[1].astype(jnp.int32)
    pad_e = E_PAD - E
    pad_ar = jnp.arange(pad_e, dtype=jnp.int32)
    src_p = jnp.concatenate([src, N + pad_ar % (N_SP - N)])
    dst_p = jnp.concatenate([dst, pad_ar % N])
    comb_p = src_p | (dst_p << 16)

    t1s = theta1[:, 0][None, :]
    t1t = theta1[:, 1][None, :]
    t2T = theta2.T
    hA = h_theta[:, :P].T
    hB = h_theta[:, P:].T
    t4T = theta4.T
    t5T = theta5.T
    t3u = theta3[:P, 0][None, :]
    t3d = theta3[P:, :]

    mu = _mu_call(x, s_v[:, None], t_v[:, None], t1s, t1t, t2T)
    agg2, degf = _get_sc_call()(comb_p, mu)
    deg0 = degf[:N, None]
    deg1 = degf[DEG_SP:DEG_SP + N, None]
    mp, smu = _b1_call(mu, agg2, agg2, deg0, deg1, hA, hB)
    return _b2_call(mp, smu, t4T, t5T, t3u, t3d)
